# baseline (speedup 1.0000x reference)
import jax
import jax.numpy as jnp
from jax import lax
from jax.experimental import pallas as pl
from jax.experimental.pallas import tpu as pltpu
from jax.experimental.pallas import tpu_sc as plsc
import dataclasses

_SC_PARAMS = pltpu.CompilerParams()
if "needs_layout_passes" in pltpu.CompilerParams.__dataclass_fields__:
    _SC_PARAMS = dataclasses.replace(_SC_PARAMS, needs_layout_passes=False)

N = 100_000
E = 3_200_000
NC, NS = 2, 16
NW = NC * NS
ROWS = 784
EPT = ROWS * 128
EPAD = NW * EPT
T = 100_352
CH = 8
NCH = ROWS // CH
TSL = T // NS
GR = T // 128


def _mesh():
    return plsc.VectorSubcoreMesh(core_axis_name="c", subcore_axis_name="s",
                                  num_cores=NC, num_subcores=NS)


def _vrsqrt(d):
    i = plsc.bitcast(d, jnp.int32)
    y = plsc.bitcast(jnp.int32(0x5F3759DF) - (i >> 1), jnp.float32)
    for _ in range(3):
        y = y * (1.5 - 0.5 * d * y * y)
    return y


def _edge_pipeline(wid, src_hbm, dst_hbm, sidx_v, didx_v, streams,
                   semi, semg, sems):
    gather = any(t is not None for t, _, _ in streams)

    def fire_idx(c, b):
        if gather:
            pltpu.async_copy(src_hbm.at[wid, pl.ds(c * CH, CH)],
                             sidx_v.at[b], semi)
        pltpu.async_copy(dst_hbm.at[wid, pl.ds(c * CH, CH)],
                         didx_v.at[b], semi)

    def wait_idx(c, b):
        if gather:
            pltpu.make_async_copy(src_hbm.at[wid, pl.ds(c * CH, CH)],
                                  sidx_v.at[b], semi).wait()
        pltpu.make_async_copy(dst_hbm.at[wid, pl.ds(c * CH, CH)],
                              didx_v.at[b], semi).wait()

    def val_ref(tab_sp, vals_v, b, j):
        return vals_v.at[b, j] if tab_sp is not None else vals_v.at[0]

    def drain_scatters(b):
        for tab_sp, vals_v, acc_sp in streams:
            for j in range(CH):
                pltpu.make_async_copy(val_ref(tab_sp, vals_v, b, j),
                                      acc_sp.at[didx_v.at[b, j]],
                                      sems).wait()

    fire_idx(0, 0)

    @pl.loop(0, NCH)
    def _(c):
        b = lax.rem(c, 2)
        wait_idx(c, b)
        for tab_sp, vals_v, _ in streams:
            if tab_sp is not None:
                for j in range(CH):
                    pltpu.async_copy(tab_sp.at[sidx_v.at[b, j]],
                                     vals_v.at[b, j], semg)

        @pl.when(c > 0)
        def _():
            drain_scatters(1 - b)

        @pl.when(c < NCH - 1)
        def _():
            fire_idx(c + 1, 1 - b)

        for tab_sp, vals_v, _ in streams:
            if tab_sp is not None:
                for j in range(CH):
                    pltpu.make_async_copy(tab_sp.at[sidx_v.at[b, j]],
                                          vals_v.at[b, j], semg).wait()
        for tab_sp, vals_v, acc_sp in streams:
            for j in range(CH):
                pltpu.async_copy(val_ref(tab_sp, vals_v, b, j),
                                 acc_sp.at[didx_v.at[b, j]], sems, add=True)

    drain_scatters((NCH - 1) % 2)


_IDX2 = pltpu.VMEM((2, CH, 128), jnp.int32)
_VAL2 = pltpu.VMEM((2, CH, 128), jnp.float32)
_NODE = pltpu.VMEM((TSL,), jnp.float32)
_TAB = pltpu.VMEM_SHARED((T,), jnp.float32)


def _sc_deg_body(dst_hbm, zeros_hbm, out_hbm,
                 didx_v, ones_v, deg_sp, semi, semg, sems):
    cid = lax.axis_index("c")
    sid = lax.axis_index("s")
    wid = cid * NS + sid
    sl = pl.ds(sid * TSL, TSL)

    for c0 in range(0, 128, 16):
        ones_v[0, pl.ds(c0, 16)] = jnp.full((16,), 1.0, jnp.float32)
    pltpu.sync_copy(zeros_hbm.at[sl], deg_sp.at[sl])
    plsc.subcore_barrier()

    _edge_pipeline(wid, None, dst_hbm, None, didx_v,
                   [(None, ones_v, deg_sp)], semi, semg, sems)

    plsc.subcore_barrier()
    pltpu.sync_copy(deg_sp.at[sl], out_hbm.at[cid, sl])


def _sc_deg(dstp, zeros_t):
    return pl.kernel(
        _sc_deg_body,
        out_type=jax.ShapeDtypeStruct((NC, T), jnp.float32),
        mesh=_mesh(),
        scratch_types=[_IDX2, pltpu.VMEM((1, 128), jnp.float32), _TAB]
        + [pltpu.SemaphoreType.DMA] * 3,
    )(dstp, zeros_t)


def _sc_agg1_body(src_hbm, dst_hbm, degp_hbm, x_hbm, zeros_hbm, out_hbm,
                  sidx_v, didx_v, vals_v, nb0, nb1, s_sp, acc_sp,
                  semi, semg, sems):
    cid = lax.axis_index("c")
    sid = lax.axis_index("s")
    wid = cid * NS + sid
    sl = pl.ds(sid * TSL, TSL)

    pltpu.sync_copy(degp_hbm.at[0, sl], nb0)
    pltpu.sync_copy(degp_hbm.at[1, sl], nb1)

    @pl.loop(0, TSL, step=16)
    def _(i):
        ds = pl.ds(i, 16)
        nb0[ds] = _vrsqrt(nb0[ds] + nb1[ds] + 1.0)

    pltpu.sync_copy(x_hbm.at[sl], nb1)

    @pl.loop(0, TSL, step=16)
    def _(i):
        ds = pl.ds(i, 16)
        nb0[ds] = nb0[ds] * nb1[ds]

    pltpu.sync_copy(nb0, s_sp.at[sl])
    pltpu.sync_copy(zeros_hbm.at[sl], acc_sp.at[sl])
    plsc.subcore_barrier()

    _edge_pipeline(wid, src_hbm, dst_hbm, sidx_v, didx_v,
                   [(s_sp, vals_v, acc_sp)], semi, semg, sems)

    plsc.subcore_barrier()
    pltpu.sync_copy(acc_sp.at[sl], out_hbm.at[cid, sl])


def _sc_agg1(srcp, dstp, degp, x_t, zeros_t):
    return pl.kernel(
        _sc_agg1_body,
        out_type=jax.ShapeDtypeStruct((NC, T), jnp.float32),
        mesh=_mesh(),
        compiler_params=_SC_PARAMS,
        scratch_types=[_IDX2, _IDX2, _VAL2, _NODE, _NODE, _TAB, _TAB]
        + [pltpu.SemaphoreType.DMA] * 3,
    )(srcp, dstp, degp, x_t, zeros_t)


def _sc_agg2_body(src_hbm, dst_hbm, degp_hbm, acc1p_hbm, x_hbm, prm_hbm,
                  zeros_hbm, out_hbm, pout_hbm,
                  sidx_v, didx_v, v0_v, v1_v, nb0, nb1, nb2, nb3, prm_v,
                  p0_sp, p1_sp, a0_sp, a1_sp, semi, semg, sems):
    cid = lax.axis_index("c")
    sid = lax.axis_index("s")
    wid = cid * NS + sid
    sl = pl.ds(sid * TSL, TSL)

    pltpu.sync_copy(prm_hbm, prm_v)
    pltpu.sync_copy(degp_hbm.at[0, sl], nb0)
    pltpu.sync_copy(degp_hbm.at[1, sl], nb1)
    pltpu.sync_copy(x_hbm.at[sl], nb2)
    pltpu.sync_copy(acc1p_hbm.at[0, sl], nb3)
    w0 = prm_v[pl.ds(0, 16)]
    w1 = prm_v[pl.ds(16, 16)]
    c0 = prm_v[pl.ds(32, 16)]
    c1 = prm_v[pl.ds(48, 16)]

    @pl.loop(0, TSL, step=16)
    def _(i):
        ds = pl.ds(i, 16)
        dis = _vrsqrt(nb0[ds] + nb1[ds] + 1.0)
        nb0[ds] = dis
        nb2[ds] = dis * nb2[ds]

    pltpu.sync_copy(acc1p_hbm.at[1, sl], nb1)

    @pl.loop(0, TSL, step=16)
    def _(i):
        ds = pl.ds(i, 16)
        dis = nb0[ds]
        u = dis * (nb3[ds] + nb1[ds] + nb2[ds])
        nb2[ds] = dis * jnp.maximum(u * w0 + c0, 0.0)
        nb3[ds] = dis * jnp.maximum(u * w1 + c1, 0.0)

    pltpu.sync_copy(nb2, p0_sp.at[sl])
    pltpu.sync_copy(nb3, p1_sp.at[sl])
    pltpu.sync_copy(zeros_hbm.at[sl], a0_sp.at[sl])
    pltpu.sync_copy(zeros_hbm.at[sl], a1_sp.at[sl])

    @pl.when(cid == 0)
    def _():
        pltpu.sync_copy(nb2, pout_hbm.at[0, sl])
        pltpu.sync_copy(nb3, pout_hbm.at[1, sl])
    plsc.subcore_barrier()

    _edge_pipeline(wid, src_hbm, dst_hbm, sidx_v, didx_v,
                   [(p0_sp, v0_v, a0_sp), (p1_sp, v1_v, a1_sp)],
                   semi, semg, sems)

    plsc.subcore_barrier()
    pltpu.sync_copy(a0_sp.at[sl], out_hbm.at[cid, 0, sl])
    pltpu.sync_copy(a1_sp.at[sl], out_hbm.at[cid, 1, sl])


def _sc_agg2(srcp, dstp, degp, acc1p, x_t, prm1b, zeros_t):
    return pl.kernel(
        _sc_agg2_body,
        out_type=[jax.ShapeDtypeStruct((NC, 2, T), jnp.float32),
                  jax.ShapeDtypeStruct((2, T), jnp.float32)],
        mesh=_mesh(),
        compiler_params=_SC_PARAMS,
        scratch_types=[_IDX2, _IDX2, _VAL2, _VAL2,
                       _NODE, _NODE, _NODE, _NODE,
                       pltpu.VMEM((64,), jnp.float32),
                       _TAB, _TAB, _TAB, _TAB]
        + [pltpu.SemaphoreType.DMA] * 3,
    )(srcp, dstp, degp, acc1p, x_t, prm1b, zeros_t)


def _tc_out_body(degp_ref, a0_ref, a1_ref, p0_ref, p1_ref, prm_ref,
                 o0_ref, o1_ref):
    dis = lax.rsqrt(degp_ref[0] + degp_ref[1] + 1.0)
    t0 = dis * (a0_ref[0] + a0_ref[1] + p0_ref[...])
    t1 = dis * (a1_ref[0] + a1_ref[1] + p1_ref[...])
    o0 = t0 * prm_ref[0] + t1 * prm_ref[2] + prm_ref[4]
    o1 = t0 * prm_ref[1] + t1 * prm_ref[3] + prm_ref[5]
    m = jnp.maximum(o0, o1)
    lse = m + jnp.log(jnp.exp(o0 - m) + jnp.exp(o1 - m))
    o0_ref[...] = o0 - lse
    o1_ref[...] = o1 - lse


def _tc_out(degp, a0, a1, p0, p1, prm2):
    return pl.pallas_call(
        _tc_out_body,
        in_specs=[pl.BlockSpec(memory_space=pltpu.MemorySpace.VMEM)] * 5
        + [pl.BlockSpec(memory_space=pltpu.MemorySpace.SMEM)],
        out_shape=[jax.ShapeDtypeStruct((GR, 128), jnp.float32)] * 2,
    )(degp, a0, a1, p0, p1, prm2)


@jax.jit
def kernel(x, edge_index, W1, b1, W2, b2):
    src = edge_index[0].astype(jnp.int32)
    dst = edge_index[1].astype(jnp.int32)
    pad = EPAD - E
    srcp = jnp.pad(src, (0, pad), constant_values=N).reshape(NW, ROWS, 128)
    dstp = jnp.pad(dst, (0, pad), constant_values=N).reshape(NW, ROWS, 128)

    zeros_t = jnp.zeros((T,), jnp.float32)
    x_t = jnp.pad(x[:, 0], (0, T - N))

    degp = _sc_deg(dstp, zeros_t)
    acc1p = _sc_agg1(srcp, dstp, degp, x_t, zeros_t)

    prm1 = jnp.concatenate([W1[0], b1]).astype(jnp.float32)
    prm1b = jnp.tile(prm1[:, None], (1, 16)).reshape(64)
    acc2p, pout = _sc_agg2(srcp, dstp, degp, acc1p, x_t, prm1b, zeros_t)

    a0 = acc2p[:, 0, :].reshape(NC, GR, 128)
    a1 = acc2p[:, 1, :].reshape(NC, GR, 128)
    prm2 = jnp.concatenate([W2[0], W2[1], b2]).astype(jnp.float32)
    o0, o1 = _tc_out(degp.reshape(NC, GR, 128), a0, a1,
                     pout[0].reshape(GR, 128), pout[1].reshape(GR, 128), prm2)

    out = jnp.stack([o0.reshape(T)[:N], o1.reshape(T)[:N]], axis=-1)
    return out


# speedup vs baseline: 1.0200x; 1.0200x over previous
import jax
import jax.numpy as jnp
from jax import lax
from jax.experimental import pallas as pl
from jax.experimental.pallas import tpu as pltpu
from jax.experimental.pallas import tpu_sc as plsc
import dataclasses

_SC_PARAMS = pltpu.CompilerParams()
if "needs_layout_passes" in pltpu.CompilerParams.__dataclass_fields__:
    _SC_PARAMS = dataclasses.replace(_SC_PARAMS, needs_layout_passes=False)

N = 100_000
E = 3_200_000
NC, NS = 2, 16
NW = NC * NS
ROWS = 784
EPT = ROWS * 128
EPAD = NW * EPT
T = 100_352
CH = 8
NCH = ROWS // CH
TSL = T // NS
GR = T // 128


def _mesh():
    return plsc.VectorSubcoreMesh(core_axis_name="c", subcore_axis_name="s",
                                  num_cores=NC, num_subcores=NS)


def _vrsqrt(d):
    i = plsc.bitcast(d, jnp.int32)
    y = plsc.bitcast(jnp.int32(0x5F3759DF) - (i >> 1), jnp.float32)
    for _ in range(2):
        y = y * (1.5 - 0.5 * d * y * y)
    return y


def _edge_pipeline(wid, src_hbm, dst_hbm, sidx_v, didx_v, streams,
                   semi, semg, sems):
    gather = any(t is not None for t, _, _ in streams)

    def fire_idx(c, b):
        if gather:
            pltpu.async_copy(src_hbm.at[wid, pl.ds(c * CH, CH)],
                             sidx_v.at[b], semi)
        pltpu.async_copy(dst_hbm.at[wid, pl.ds(c * CH, CH)],
                         didx_v.at[b], semi)

    def wait_idx(c, b):
        if gather:
            pltpu.make_async_copy(src_hbm.at[wid, pl.ds(c * CH, CH)],
                                  sidx_v.at[b], semi).wait()
        pltpu.make_async_copy(dst_hbm.at[wid, pl.ds(c * CH, CH)],
                              didx_v.at[b], semi).wait()

    def val_ref(tab_sp, vals_v, b, j):
        return vals_v.at[b, j] if tab_sp is not None else vals_v.at[0]

    def drain_scatters(b):
        for tab_sp, vals_v, acc_sp in streams:
            for j in range(CH):
                pltpu.make_async_copy(val_ref(tab_sp, vals_v, b, j),
                                      acc_sp.at[didx_v.at[b, j]],
                                      sems).wait()

    fire_idx(0, 0)

    @pl.loop(0, NCH)
    def _(c):
        b = lax.rem(c, 2)
        wait_idx(c, b)
        for tab_sp, vals_v, _ in streams:
            if tab_sp is not None:
                for j in range(CH):
                    pltpu.async_copy(tab_sp.at[sidx_v.at[b, j]],
                                     vals_v.at[b, j], semg)

        @pl.when(c > 0)
        def _():
            drain_scatters(1 - b)

        @pl.when(c < NCH - 1)
        def _():
            fire_idx(c + 1, 1 - b)

        for tab_sp, vals_v, _ in streams:
            if tab_sp is not None:
                for j in range(CH):
                    pltpu.make_async_copy(tab_sp.at[sidx_v.at[b, j]],
                                          vals_v.at[b, j], semg).wait()
        for tab_sp, vals_v, acc_sp in streams:
            for j in range(CH):
                pltpu.async_copy(val_ref(tab_sp, vals_v, b, j),
                                 acc_sp.at[didx_v.at[b, j]], sems, add=True)

    drain_scatters((NCH - 1) % 2)


_IDX2 = pltpu.VMEM((2, CH, 128), jnp.int32)
_VAL2 = pltpu.VMEM((2, CH, 128), jnp.float32)
_NODE = pltpu.VMEM((TSL,), jnp.float32)
_TAB = pltpu.VMEM_SHARED((T,), jnp.float32)


def _sc_deg_body(dst_hbm, zeros_hbm, out_hbm,
                 didx_v, ones_v, deg_sp, semi, semg, sems):
    cid = lax.axis_index("c")
    sid = lax.axis_index("s")
    wid = cid * NS + sid
    sl = pl.ds(sid * TSL, TSL)

    for c0 in range(0, 128, 16):
        ones_v[0, pl.ds(c0, 16)] = jnp.full((16,), 1.0, jnp.float32)
    pltpu.sync_copy(zeros_hbm.at[sl], deg_sp.at[sl])
    plsc.subcore_barrier()

    _edge_pipeline(wid, None, dst_hbm, None, didx_v,
                   [(None, ones_v, deg_sp)], semi, semg, sems)

    plsc.subcore_barrier()
    pltpu.sync_copy(deg_sp.at[sl], out_hbm.at[cid, sl])


def _sc_deg(dstp, zeros_t):
    return pl.kernel(
        _sc_deg_body,
        out_type=jax.ShapeDtypeStruct((NC, T), jnp.float32),
        mesh=_mesh(),
        scratch_types=[_IDX2, pltpu.VMEM((1, 128), jnp.float32), _TAB]
        + [pltpu.SemaphoreType.DMA] * 3,
    )(dstp, zeros_t)


def _sc_agg1_body(src_hbm, dst_hbm, degp_hbm, x_hbm, zeros_hbm, out_hbm,
                  sidx_v, didx_v, vals_v, nb0, nb1, nb2, s_sp, acc_sp,
                  semi, semg, sems):
    cid = lax.axis_index("c")
    sid = lax.axis_index("s")
    wid = cid * NS + sid
    sl = pl.ds(sid * TSL, TSL)

    pltpu.async_copy(degp_hbm.at[0, sl], nb0, semg)
    pltpu.async_copy(degp_hbm.at[1, sl], nb1, semg)
    pltpu.async_copy(x_hbm.at[sl], nb2, semg)
    for ref, hbm in ((nb0, degp_hbm.at[0, sl]), (nb1, degp_hbm.at[1, sl]),
                     (nb2, x_hbm.at[sl])):
        pltpu.make_async_copy(hbm, ref, semg).wait()

    @pl.loop(0, TSL, step=16, unroll=8)
    def _(i):
        ds = pl.ds(i, 16)
        nb0[ds] = _vrsqrt(nb0[ds] + nb1[ds] + 1.0) * nb2[ds]

    pltpu.sync_copy(nb0, s_sp.at[sl])
    pltpu.sync_copy(zeros_hbm.at[sl], acc_sp.at[sl])
    plsc.subcore_barrier()

    _edge_pipeline(wid, src_hbm, dst_hbm, sidx_v, didx_v,
                   [(s_sp, vals_v, acc_sp)], semi, semg, sems)

    plsc.subcore_barrier()
    pltpu.sync_copy(acc_sp.at[sl], out_hbm.at[cid, sl])


def _sc_agg1(srcp, dstp, degp, x_t, zeros_t):
    return pl.kernel(
        _sc_agg1_body,
        out_type=jax.ShapeDtypeStruct((NC, T), jnp.float32),
        mesh=_mesh(),
        compiler_params=_SC_PARAMS,
        scratch_types=[_IDX2, _IDX2, _VAL2, _NODE, _NODE, _NODE, _TAB, _TAB]
        + [pltpu.SemaphoreType.DMA] * 3,
    )(srcp, dstp, degp, x_t, zeros_t)


def _sc_agg2_body(src_hbm, dst_hbm, degp_hbm, acc1p_hbm, x_hbm, prm_hbm,
                  zeros_hbm, out_hbm, pout_hbm,
                  sidx_v, didx_v, v0_v, v1_v, nb0, nb1, nb2, nb3, nb4, prm_v,
                  p0_sp, p1_sp, a0_sp, a1_sp, semi, semg, sems):
    cid = lax.axis_index("c")
    sid = lax.axis_index("s")
    wid = cid * NS + sid
    sl = pl.ds(sid * TSL, TSL)

    pltpu.sync_copy(prm_hbm, prm_v)
    loads = ((nb0, degp_hbm.at[0, sl]), (nb1, degp_hbm.at[1, sl]),
             (nb2, x_hbm.at[sl]), (nb3, acc1p_hbm.at[0, sl]),
             (nb4, acc1p_hbm.at[1, sl]))
    for ref, hbm in loads:
        pltpu.async_copy(hbm, ref, semg)
    for ref, hbm in loads:
        pltpu.make_async_copy(hbm, ref, semg).wait()
    w0 = prm_v[pl.ds(0, 16)]
    w1 = prm_v[pl.ds(16, 16)]
    c0 = prm_v[pl.ds(32, 16)]
    c1 = prm_v[pl.ds(48, 16)]

    @pl.loop(0, TSL, step=16, unroll=8)
    def _(i):
        ds = pl.ds(i, 16)
        dis = _vrsqrt(nb0[ds] + nb1[ds] + 1.0)
        u = dis * (nb3[ds] + nb4[ds] + dis * nb2[ds])
        nb2[ds] = dis * jnp.maximum(u * w0 + c0, 0.0)
        nb3[ds] = dis * jnp.maximum(u * w1 + c1, 0.0)

    pltpu.sync_copy(nb2, p0_sp.at[sl])
    pltpu.sync_copy(nb3, p1_sp.at[sl])
    pltpu.sync_copy(zeros_hbm.at[sl], a0_sp.at[sl])
    pltpu.sync_copy(zeros_hbm.at[sl], a1_sp.at[sl])

    @pl.when(cid == 0)
    def _():
        pltpu.sync_copy(nb2, pout_hbm.at[0, sl])
        pltpu.sync_copy(nb3, pout_hbm.at[1, sl])
    plsc.subcore_barrier()

    _edge_pipeline(wid, src_hbm, dst_hbm, sidx_v, didx_v,
                   [(p0_sp, v0_v, a0_sp), (p1_sp, v1_v, a1_sp)],
                   semi, semg, sems)

    plsc.subcore_barrier()
    pltpu.sync_copy(a0_sp.at[sl], out_hbm.at[cid, 0, sl])
    pltpu.sync_copy(a1_sp.at[sl], out_hbm.at[cid, 1, sl])


def _sc_agg2(srcp, dstp, degp, acc1p, x_t, prm1b, zeros_t):
    return pl.kernel(
        _sc_agg2_body,
        out_type=[jax.ShapeDtypeStruct((NC, 2, T), jnp.float32),
                  jax.ShapeDtypeStruct((2, T), jnp.float32)],
        mesh=_mesh(),
        compiler_params=_SC_PARAMS,
        scratch_types=[_IDX2, _IDX2, _VAL2, _VAL2,
                       _NODE, _NODE, _NODE, _NODE, _NODE,
                       pltpu.VMEM((64,), jnp.float32),
                       _TAB, _TAB, _TAB, _TAB]
        + [pltpu.SemaphoreType.DMA] * 3,
    )(srcp, dstp, degp, acc1p, x_t, prm1b, zeros_t)


def _tc_out_body(degp_ref, a0_ref, a1_ref, p0_ref, p1_ref, prm_ref,
                 o0_ref, o1_ref):
    dis = lax.rsqrt(degp_ref[0] + degp_ref[1] + 1.0)
    t0 = dis * (a0_ref[0] + a0_ref[1] + p0_ref[...])
    t1 = dis * (a1_ref[0] + a1_ref[1] + p1_ref[...])
    o0 = t0 * prm_ref[0] + t1 * prm_ref[2] + prm_ref[4]
    o1 = t0 * prm_ref[1] + t1 * prm_ref[3] + prm_ref[5]
    m = jnp.maximum(o0, o1)
    lse = m + jnp.log(jnp.exp(o0 - m) + jnp.exp(o1 - m))
    o0_ref[...] = o0 - lse
    o1_ref[...] = o1 - lse


def _tc_out(degp, a0, a1, p0, p1, prm2):
    return pl.pallas_call(
        _tc_out_body,
        in_specs=[pl.BlockSpec(memory_space=pltpu.MemorySpace.VMEM)] * 5
        + [pl.BlockSpec(memory_space=pltpu.MemorySpace.SMEM)],
        out_shape=[jax.ShapeDtypeStruct((GR, 128), jnp.float32)] * 2,
    )(degp, a0, a1, p0, p1, prm2)


@jax.jit
def kernel(x, edge_index, W1, b1, W2, b2):
    src = edge_index[0].astype(jnp.int32)
    dst = edge_index[1].astype(jnp.int32)
    pad = EPAD - E
    srcp = jnp.pad(src, (0, pad), constant_values=N).reshape(NW, ROWS, 128)
    dstp = jnp.pad(dst, (0, pad), constant_values=N).reshape(NW, ROWS, 128)

    zeros_t = jnp.zeros((T,), jnp.float32)
    x_t = jnp.pad(x[:, 0], (0, T - N))

    degp = _sc_deg(dstp, zeros_t)
    acc1p = _sc_agg1(srcp, dstp, degp, x_t, zeros_t)

    prm1 = jnp.concatenate([W1[0], b1]).astype(jnp.float32)
    prm1b = jnp.tile(prm1[:, None], (1, 16)).reshape(64)
    acc2p, pout = _sc_agg2(srcp, dstp, degp, acc1p, x_t, prm1b, zeros_t)

    a0 = acc2p[:, 0, :].reshape(NC, GR, 128)
    a1 = acc2p[:, 1, :].reshape(NC, GR, 128)
    prm2 = jnp.concatenate([W2[0], W2[1], b2]).astype(jnp.float32)
    o0, o1 = _tc_out(degp.reshape(NC, GR, 128), a0, a1,
                     pout[0].reshape(GR, 128), pout[1].reshape(GR, 128), prm2)

    out = jnp.stack([o0.reshape(T)[:N], o1.reshape(T)[:N]], axis=-1)
    return out
